# initial kernel scaffold (unmeasured)
import functools

import jax
import jax.numpy as jnp
from jax import lax
from jax.experimental import pallas as pl
from jax.experimental.pallas import tpu as pltpu

N_DEV = 32
W_CORR = 64


def kernel(x, A, B, C):
    Bb, S, D = x.shape
    N = A.shape[1]

    def body(x_ref, A_ref, B_ref, C_ref, out_ref, sbuf, rbuf, send_sem, recv_sem):
        my = lax.axis_index("i")
        left = lax.rem(my + N_DEV - 1, N_DEV)
        right = lax.rem(my + 1, N_DEV)

        barrier_sem = pltpu.get_barrier_semaphore()
        for nbr in (left, right):
            pl.semaphore_signal(
                barrier_sem, inc=1,
                device_id=(nbr,), device_id_type=pl.DeviceIdType.MESH,
            )
        pl.semaphore_wait(barrier_sem, 2)

        An = A_ref[...].T
        dA = jnp.exp(An)[None]

        def step(t, h):
            xt = x_ref[:, pl.ds(t, 1), :]
            bt = B_ref[:, pl.ds(t, 1), :]
            ct = C_ref[:, pl.ds(t, 1), :]
            h = h * dA + xt * jnp.transpose(bt, (0, 2, 1))
            y = jnp.sum(h * jnp.transpose(ct, (0, 2, 1)), axis=1, keepdims=True)
            out_ref[:, pl.ds(t, 1), :] = y
            return h

        h_fin = lax.fori_loop(0, S, step, jnp.zeros((Bb, N, D), jnp.float32))
        sbuf[...] = h_fin

        rdma = pltpu.make_async_remote_copy(
            src_ref=sbuf,
            dst_ref=rbuf,
            send_sem=send_sem,
            recv_sem=recv_sem,
            device_id=(right,),
            device_id_type=pl.DeviceIdType.MESH,
        )
        rdma.start()
        rdma.wait()

        h_prev = rbuf[...] * jnp.where(my == 0, 0.0, 1.0)
        tvals = lax.broadcasted_iota(jnp.float32, (W_CORR, N, D), 0) + 1.0
        P = jnp.exp(An[None] * tvals)
        for b in range(Bb):
            G = P * h_prev[b][None]
            cb = jnp.transpose(C_ref[b, :W_CORR, :])[None]
            yc = jnp.sum(G * jnp.transpose(cb, (2, 1, 0)), axis=1)
            out_ref[b, :W_CORR, :] = out_ref[b, :W_CORR, :] + yc

        @functools.partial(
            pl.run_scoped, second_barrier=pltpu.SemaphoreType.REGULAR
        )
        def _(second_barrier):
            for nbr in (left, right):
                pl.semaphore_signal(
                    second_barrier, inc=1,
                    device_id=(nbr,), device_id_type=pl.DeviceIdType.MESH,
                )
            pl.semaphore_wait(second_barrier, 2)

    return pl.pallas_call(
        body,
        out_shape=jax.ShapeDtypeStruct((Bb, S, D), jnp.float32),
        in_specs=[pl.BlockSpec(memory_space=pltpu.VMEM)] * 4,
        out_specs=pl.BlockSpec(memory_space=pltpu.VMEM),
        scratch_shapes=[
            pltpu.VMEM((Bb, N, D), jnp.float32),
            pltpu.VMEM((Bb, N, D), jnp.float32),
            pltpu.SemaphoreType.DMA,
            pltpu.SemaphoreType.DMA,
        ],
        compiler_params=pltpu.CompilerParams(collective_id=0),
    )(x, A, B, C)


# baseline (device time: 60230 ns/iter reference)
import functools

import jax
import jax.numpy as jnp
from jax import lax
from jax.experimental import pallas as pl
from jax.experimental.pallas import tpu as pltpu

N_DEV = 32
W_CORR = 64


def kernel(x, A, B, C):
    Bb, S, D = x.shape
    N = A.shape[1]

    def body(x_ref, A_ref, B_ref, C_ref, out_ref, sbuf, rbuf, send_sem, recv_sem):
        my = lax.axis_index("i")
        left = lax.rem(my + N_DEV - 1, N_DEV)
        right = lax.rem(my + 1, N_DEV)

        barrier_sem = pltpu.get_barrier_semaphore()
        for nbr in (left, right):
            pl.semaphore_signal(
                barrier_sem, inc=1,
                device_id=(nbr,), device_id_type=pl.DeviceIdType.MESH,
            )
        pl.semaphore_wait(barrier_sem, 2)

        An = A_ref[...].T
        dA = jnp.exp(An)[None]

        def step(t, h):
            xt = x_ref[:, pl.ds(t, 1), :]
            bt = B_ref[:, pl.ds(t, 1), :]
            ct = C_ref[:, pl.ds(t, 1), :]
            h = h * dA + xt * jnp.transpose(bt, (0, 2, 1))
            y = jnp.sum(h * jnp.transpose(ct, (0, 2, 1)), axis=1, keepdims=True)
            out_ref[:, pl.ds(t, 1), :] = y
            return h

        h_fin = lax.fori_loop(0, S, step, jnp.zeros((Bb, N, D), jnp.float32))
        sbuf[...] = h_fin

        rdma = pltpu.make_async_remote_copy(
            src_ref=sbuf,
            dst_ref=rbuf,
            send_sem=send_sem,
            recv_sem=recv_sem,
            device_id=(right,),
            device_id_type=pl.DeviceIdType.MESH,
        )
        rdma.start()
        rdma.wait()

        h_prev = rbuf[...] * jnp.where(my == 0, 0.0, 1.0)
        tvals = (
            lax.broadcasted_iota(jnp.int32, (W_CORR, N, D), 0) + 1
        ).astype(jnp.float32)
        P = jnp.exp(An[None] * tvals)
        for b in range(Bb):
            G = P * h_prev[b][None]
            cb = jnp.transpose(C_ref[b, :W_CORR, :])[None]
            yc = jnp.sum(G * jnp.transpose(cb, (2, 1, 0)), axis=1)
            out_ref[b, :W_CORR, :] = out_ref[b, :W_CORR, :] + yc

        @functools.partial(
            pl.run_scoped, second_barrier=pltpu.SemaphoreType.REGULAR
        )
        def _(second_barrier):
            for nbr in (left, right):
                pl.semaphore_signal(
                    second_barrier, inc=1,
                    device_id=(nbr,), device_id_type=pl.DeviceIdType.MESH,
                )
            pl.semaphore_wait(second_barrier, 2)

    return pl.pallas_call(
        body,
        out_shape=jax.ShapeDtypeStruct((Bb, S, D), jnp.float32),
        in_specs=[pl.BlockSpec(memory_space=pltpu.VMEM)] * 4,
        out_specs=pl.BlockSpec(memory_space=pltpu.VMEM),
        scratch_shapes=[
            pltpu.VMEM((Bb, N, D), jnp.float32),
            pltpu.VMEM((Bb, N, D), jnp.float32),
            pltpu.SemaphoreType.DMA,
            pltpu.SemaphoreType.DMA,
        ],
        compiler_params=pltpu.CompilerParams(collective_id=0),
    )(x, A, B, C)


# device time: 14071 ns/iter; 4.2804x vs baseline; 4.2804x over previous
import functools

import jax
import jax.numpy as jnp
from jax import lax
from jax.experimental import pallas as pl
from jax.experimental.pallas import tpu as pltpu

N_DEV = 32
W_SEND = 16
W_CORR = 16
T_BLK = 16


def kernel(x, A, B, C):
    Bb, S, D = x.shape
    N = A.shape[1]

    def body(x_ref, A_ref, B_ref, C_ref, out_ref, sbuf, rbuf, send_sem, recv_sem):
        my = lax.axis_index("i")
        left = lax.rem(my + N_DEV - 1, N_DEV)
        right = lax.rem(my + 1, N_DEV)

        barrier_sem = pltpu.get_barrier_semaphore()
        pl.semaphore_signal(
            barrier_sem, inc=1,
            device_id=(left,), device_id_type=pl.DeviceIdType.MESH,
        )
        pl.semaphore_wait(barrier_sem, 1)

        An = A_ref[...].T
        dA = jnp.exp(An)[None]
        dA_h = dA.astype(jnp.bfloat16)

        def scan_block(h, t0_dyn, with_y):
            xb = x_ref[:, pl.ds(t0_dyn, T_BLK), :].astype(jnp.bfloat16)
            bbT = jnp.transpose(
                B_ref[:, pl.ds(t0_dyn, T_BLK), :].astype(jnp.bfloat16),
                (0, 2, 1),
            )
            cbT = (
                jnp.transpose(
                    C_ref[:, pl.ds(t0_dyn, T_BLK), :].astype(jnp.bfloat16),
                    (0, 2, 1),
                )
                if with_y else None
            )
            ys = []
            for j in range(T_BLK):
                h = h * dA_h + xb[:, j:j + 1, :] * bbT[:, :, j:j + 1]
                if with_y:
                    ys.append(jnp.sum(h * cbT[:, :, j:j + 1], axis=1))
            if with_y:
                out_ref[:, pl.ds(t0_dyn, T_BLK), :] = jnp.stack(
                    ys, axis=1
                ).astype(jnp.float32)
            return h

        h_send = jnp.zeros((Bb, N, D), jnp.bfloat16)
        for k in range(W_SEND // T_BLK):
            h_send = scan_block(h_send, S - W_SEND + k * T_BLK, with_y=False)
        sbuf[...] = h_send

        rdma = pltpu.make_async_remote_copy(
            src_ref=sbuf,
            dst_ref=rbuf,
            send_sem=send_sem,
            recv_sem=recv_sem,
            device_id=(right,),
            device_id_type=pl.DeviceIdType.MESH,
        )
        rdma.start()

        def blk(i, h):
            return scan_block(h, i * T_BLK, with_y=True)

        lax.fori_loop(0, S // T_BLK, blk, jnp.zeros((Bb, N, D), jnp.bfloat16))

        rdma.wait_recv()
        h_prev = rbuf[...].astype(jnp.float32) * jnp.where(my == 0, 0.0, 1.0)
        tvals = (
            lax.broadcasted_iota(jnp.int32, (W_CORR, N, D), 0) + 1
        ).astype(jnp.float32)
        P = jnp.exp(An[None] * tvals)
        for b in range(Bb):
            G = P * h_prev[b][None]
            cw = C_ref[b, :W_CORR, :][:, :, None]
            yc = jnp.sum(G * cw, axis=1)
            out_ref[b, :W_CORR, :] = out_ref[b, :W_CORR, :] + yc

        rdma.wait_send()

        @functools.partial(
            pl.run_scoped, second_barrier=pltpu.SemaphoreType.REGULAR
        )
        def _(second_barrier):
            pl.semaphore_signal(
                second_barrier, inc=1,
                device_id=(left,), device_id_type=pl.DeviceIdType.MESH,
            )
            pl.semaphore_wait(second_barrier, 1)

    return pl.pallas_call(
        body,
        out_shape=jax.ShapeDtypeStruct((Bb, S, D), jnp.float32),
        in_specs=[pl.BlockSpec(memory_space=pltpu.VMEM)] * 4,
        out_specs=pl.BlockSpec(memory_space=pltpu.VMEM),
        scratch_shapes=[
            pltpu.VMEM((Bb, N, D), jnp.bfloat16),
            pltpu.VMEM((Bb, N, D), jnp.bfloat16),
            pltpu.SemaphoreType.DMA,
            pltpu.SemaphoreType.DMA,
        ],
        compiler_params=pltpu.CompilerParams(collective_id=0),
    )(x, A, B, C)
